# baseline (speedup 1.0000x reference)
import functools

import jax
import jax.numpy as jnp
from jax import lax
from jax.experimental import pallas as pl
from jax.experimental.pallas import tpu as pltpu
from jax.experimental.pallas import tpu_sc as plsc

THRESH = 0.1
BM = 256
NC, NS = 2, 16
NW = NC * NS


def _route_body(x_ref, gw1_ref, gb1_ref, gw2_ref, gb2_ref, alpha_ref,
                gwout_ref, destcat_ref, wt0_ref, wt1_ref, s1_ref, s2_ref,
                benb_ref, g_max, g_cap):
    B = x_ref.shape[0]
    E = gw2_ref.shape[1]
    xb = x_ref[...]
    h = jnp.dot(xb, gw1_ref[...], preferred_element_type=jnp.float32)
    h = h + gb1_ref[...]
    h = h * jax.nn.sigmoid(h)
    logits = jnp.dot(h, gw2_ref[...], preferred_element_type=jnp.float32)
    logits = logits + gb2_ref[...]
    m = jnp.max(logits, axis=1, keepdims=True)
    p = jnp.exp(logits - m)
    gw = p / jnp.sum(p, axis=1, keepdims=True)
    gwout_ref[...] = gw

    lane = jax.lax.broadcasted_iota(jnp.int32, (B, E), 1)
    cols = []
    for ee in range(E):
        ge = gw[:, ee:ee + 1]
        gt = jnp.sum((gw > ge).astype(jnp.int32), axis=1, keepdims=True)
        eqb = jnp.sum(((gw == ge) & (lane < ee)).astype(jnp.int32),
                      axis=1, keepdims=True)
        fire = ((gt + eqb) < 2) & (ge > THRESH)
        cols.append(jnp.where(fire, ge, 0.0))
    w = jnp.concatenate(cols, axis=1)
    fire = w > 0.0
    fire_f = fire.astype(jnp.float32)

    RB = 512
    pos_blocks = []
    for rb in range(B // RB):
        ri = jax.lax.broadcasted_iota(jnp.int32, (RB, B), 0) + rb * RB
        ci = jax.lax.broadcasted_iota(jnp.int32, (RB, B), 1)
        lblk = (ci < ri).astype(jnp.float32)
        pos_blocks.append(jnp.dot(lblk, fire_f,
                                  preferred_element_type=jnp.float32))
    pos = jnp.concatenate(pos_blocks, axis=0)
    c_e = jnp.sum(fire_f, axis=0, keepdims=True)
    nb_e = jnp.floor_divide(c_e.astype(jnp.int32) + BM - 1, BM)
    ui = (jax.lax.broadcasted_iota(jnp.int32, (E, E), 0)
          <= jax.lax.broadcasted_iota(jnp.int32, (E, E), 1))
    nb_cum = jnp.dot(nb_e.astype(jnp.float32), ui.astype(jnp.float32),
                     preferred_element_type=jnp.float32)
    base = ((nb_cum - nb_e.astype(jnp.float32)) * BM)
    dest = (base + pos).astype(jnp.int32)

    big = jnp.int32(g_cap)
    dmin = jnp.min(jnp.where(fire, dest, big), axis=1, keepdims=True)
    dmax = jnp.max(jnp.where(fire, dest, -1), axis=1, keepdims=True)
    nf = jnp.sum(fire.astype(jnp.int32), axis=1, keepdims=True)
    w_min = jnp.sum(jnp.where(dest == dmin, w, 0.0), axis=1, keepdims=True)
    w_max = jnp.sum(jnp.where(dest == dmax, w, 0.0), axis=1, keepdims=True)
    toki = jax.lax.broadcasted_iota(jnp.int32, (B, 1), 0)
    dummy = (g_cap - BM) + (toki & (BM - 1))
    dest0 = jnp.where(nf >= 1, dmin, dummy)
    dest1 = jnp.where(nf >= 2, dmax, dummy)
    wt0 = w_min
    wt1 = jnp.where(nf >= 2, w_max, 0.0)
    destcat_ref[0:B, :] = dest0
    destcat_ref[B:2 * B, :] = dest1
    wt0_ref[...] = wt0
    wt1_ref[...] = wt1
    tw = wt0 + wt1
    fired = nf >= 1
    stw = jnp.where(fired, tw, 1.0)
    a = alpha_ref[0, 0]
    s1_ref[...] = a / stw
    s2_ref[...] = jnp.where(fired, 1.0 - a, 1.0)

    gi = jax.lax.broadcasted_iota(jnp.int32, (32, E), 0)
    nbc_i = nb_cum.astype(jnp.int32)
    be = jnp.sum((gi >= nbc_i).astype(jnp.int32), axis=1, keepdims=True)
    be = jnp.minimum(be, E - 1)
    rowi = jax.lax.broadcasted_iota(jnp.int32, (32, 1), 0)
    benb_ref[...] = jnp.where(rowi == g_max, nbc_i[:, E - 1:E], be)


def _route_call(x, gate_w1, gate_b1, gate_w2, gate_b2, alpha, g_max, g_cap):
    B, H = x.shape
    H2 = gate_w1.shape[1]
    E = gate_w2.shape[1]
    return pl.pallas_call(
        functools.partial(_route_body, g_max=g_max, g_cap=g_cap),
        in_specs=[
            pl.BlockSpec((B, H), lambda: (0, 0)),
            pl.BlockSpec((H, H2), lambda: (0, 0)),
            pl.BlockSpec((1, H2), lambda: (0, 0)),
            pl.BlockSpec((H2, E), lambda: (0, 0)),
            pl.BlockSpec((1, E), lambda: (0, 0)),
            pl.BlockSpec(memory_space=pltpu.SMEM),
        ],
        out_specs=[
            pl.BlockSpec((B, E), lambda: (0, 0)),
            pl.BlockSpec((2 * B, 1), lambda: (0, 0)),
            pl.BlockSpec((B, 1), lambda: (0, 0)),
            pl.BlockSpec((B, 1), lambda: (0, 0)),
            pl.BlockSpec((B, 1), lambda: (0, 0)),
            pl.BlockSpec((B, 1), lambda: (0, 0)),
            pl.BlockSpec((32, 1), lambda: (0, 0)),
        ],
        out_shape=[
            jax.ShapeDtypeStruct((B, E), jnp.float32),
            jax.ShapeDtypeStruct((2 * B, 1), jnp.int32),
            jax.ShapeDtypeStruct((B, 1), jnp.float32),
            jax.ShapeDtypeStruct((B, 1), jnp.float32),
            jax.ShapeDtypeStruct((B, 1), jnp.float32),
            jax.ShapeDtypeStruct((B, 1), jnp.float32),
            jax.ShapeDtypeStruct((32, 1), jnp.int32),
        ],
    )(x, gate_w1, gate_b1.reshape(1, H2), gate_w2, gate_b2.reshape(1, E),
      alpha)


def _sc_scatter_rows(x, idx3, n_out):
    B, D = x.shape
    tpw = B // NW
    mesh = plsc.VectorSubcoreMesh(core_axis_name="c", subcore_axis_name="s")

    @functools.partial(
        pl.kernel, mesh=mesh,
        out_type=jax.ShapeDtypeStruct((n_out, D), jnp.float32),
        scratch_types=[
            pltpu.VMEM((2, tpw), jnp.int32),
            pltpu.VMEM((tpw, D), jnp.float32),
            pltpu.SemaphoreType.DMA,
            pltpu.SemaphoreType.DMA,
        ],
    )
    def k(x_hbm, idx_hbm, out_hbm, idx_v, buf, sem0, sem1):
        wid = lax.axis_index("s") * NC + lax.axis_index("c")
        pltpu.sync_copy(idx_hbm.at[wid], idx_v)
        pltpu.sync_copy(x_hbm.at[pl.ds(wid * tpw, tpw)], buf)
        c0 = pltpu.async_copy(buf, out_hbm.at[idx_v.at[0]], sem0)
        c1 = pltpu.async_copy(buf, out_hbm.at[idx_v.at[1]], sem1)
        c0.wait()
        c1.wait()

    return k(x, idx3)


def _sc_gather(table, idx, chunk):
    n_rows = idx.shape[0]
    D = table.shape[1]
    rows_per_tile = n_rows // NW
    n_chunks = rows_per_tile // chunk
    mesh = plsc.VectorSubcoreMesh(core_axis_name="c", subcore_axis_name="s")

    @functools.partial(
        pl.kernel, mesh=mesh,
        out_type=jax.ShapeDtypeStruct((n_rows, D), jnp.float32),
        scratch_types=[
            pltpu.VMEM((rows_per_tile,), jnp.int32),
            pltpu.VMEM((chunk, D), jnp.float32),
            pltpu.VMEM((chunk, D), jnp.float32),
            pltpu.SemaphoreType.DMA,
            pltpu.SemaphoreType.DMA,
        ],
    )
    def k(table_hbm, idx_hbm, out_hbm, idx_v, buf0, buf1, sem_g, sem_s):
        wid = lax.axis_index("s") * NC + lax.axis_index("c")
        base = wid * rows_per_tile
        bufs = (buf0, buf1)
        pltpu.sync_copy(idx_hbm.at[pl.ds(base, rows_per_tile)], idx_v)
        gathers = []
        stores = []
        gathers.append(pltpu.async_copy(
            table_hbm.at[idx_v.at[pl.ds(0, chunk)]], bufs[0], sem_g))
        for c in range(n_chunks):
            gathers[c].wait()
            if c + 1 < n_chunks:
                if c >= 1:
                    stores[c - 1].wait()
                gathers.append(pltpu.async_copy(
                    table_hbm.at[idx_v.at[pl.ds((c + 1) * chunk, chunk)]],
                    bufs[(c + 1) % 2], sem_g))
            stores.append(pltpu.async_copy(
                bufs[c % 2], out_hbm.at[pl.ds(base + c * chunk, chunk)],
                sem_s))
        stores[n_chunks - 2].wait()
        stores[n_chunks - 1].wait()

    return k(table, idx)


def _expert_body(be_ref, nb_ref, xs_ref,
                 ew1_ref, eb1_ref, ew2_ref, eb2_ref, pw_ref, zs_ref):
    g = pl.program_id(0)

    @pl.when(g < nb_ref[0])
    def _compute():
        xb = xs_ref[...]
        h1 = jnp.dot(xb, ew1_ref[0], preferred_element_type=jnp.float32)
        h1 = h1 + eb1_ref[0]
        h1 = h1 * jax.nn.sigmoid(h1)
        eo = jnp.dot(h1, ew2_ref[0], preferred_element_type=jnp.float32)
        eo = eo + eb2_ref[0]
        zs_ref[...] = jnp.dot(eo, pw_ref[0], preferred_element_type=jnp.float32)

    @pl.when(g >= nb_ref[0])
    def _zero():
        zs_ref[...] = jnp.zeros_like(zs_ref)


def _expert_call(xs, expert_w1, expert_b1, expert_w2, expert_b2, proj_w,
                 block_expert, nb_arr, g_max):
    G_CAP, H = xs.shape
    E, _, F = expert_w1.shape
    grid_spec = pltpu.PrefetchScalarGridSpec(
        num_scalar_prefetch=2,
        grid=(g_max,),
        in_specs=[
            pl.BlockSpec((BM, H), lambda g, be, nb: (g, 0)),
            pl.BlockSpec((1, H, F), lambda g, be, nb: (be[g], 0, 0)),
            pl.BlockSpec((1, 1, F), lambda g, be, nb: (be[g], 0, 0)),
            pl.BlockSpec((1, F, H), lambda g, be, nb: (be[g], 0, 0)),
            pl.BlockSpec((1, 1, H), lambda g, be, nb: (be[g], 0, 0)),
            pl.BlockSpec((1, H, H), lambda g, be, nb: (be[g], 0, 0)),
        ],
        out_specs=pl.BlockSpec((BM, H), lambda g, be, nb: (g, 0)),
    )
    return pl.pallas_call(
        _expert_body,
        grid_spec=grid_spec,
        out_shape=jax.ShapeDtypeStruct((G_CAP, H), jnp.float32),
    )(block_expert, nb_arr, xs,
      expert_w1, expert_b1.reshape(E, 1, F), expert_w2,
      expert_b2.reshape(E, 1, H), proj_w)


def _combine_body(g0_ref, g1_ref, x_ref, wt0_ref, wt1_ref, s1_ref, s2_ref,
                  out_ref):
    acc = wt0_ref[...] * g0_ref[...] + wt1_ref[...] * g1_ref[...]
    out_ref[...] = s1_ref[...] * acc + s2_ref[...] * x_ref[...]


def _combine_call(gath, x, wt0, wt1, s1, s2):
    B, H = x.shape
    BMc = 512
    MB = B // BMc
    return pl.pallas_call(
        _combine_body,
        grid=(MB,),
        in_specs=[
            pl.BlockSpec((BMc, H), lambda mb: (mb, 0)),
            pl.BlockSpec((BMc, H), lambda mb, _MB=MB: (mb + _MB, 0)),
            pl.BlockSpec((BMc, H), lambda mb: (mb, 0)),
            pl.BlockSpec((BMc, 1), lambda mb: (mb, 0)),
            pl.BlockSpec((BMc, 1), lambda mb: (mb, 0)),
            pl.BlockSpec((BMc, 1), lambda mb: (mb, 0)),
            pl.BlockSpec((BMc, 1), lambda mb: (mb, 0)),
        ],
        out_specs=pl.BlockSpec((BMc, H), lambda mb: (mb, 0)),
        out_shape=jax.ShapeDtypeStruct((B, H), jnp.float32),
    )(gath, gath, x, wt0, wt1, s1, s2)


def kernel(x, gate_w1, gate_b1, gate_w2, gate_b2,
           expert_w1, expert_b1, expert_w2, expert_b2, proj_w, blend):
    B, H = x.shape
    E = gate_w2.shape[1]
    G_MAX = (2 * B) // BM + E + 1
    G_CAP = G_MAX * BM

    alpha = jax.nn.sigmoid(blend).reshape(1, 1).astype(jnp.float32)
    (gate_weights, destcat, wt0, wt1, s1, s2, benb) = _route_call(
        x, gate_w1, gate_b1, gate_w2, gate_b2, alpha, G_MAX, G_CAP)

    be_arr = benb[:G_MAX, 0]
    nb_arr = benb[G_MAX:G_MAX + 1, 0]
    destflat = destcat.reshape(2 * B)
    idx3 = destcat.reshape(2, NW, B // NW).transpose(1, 0, 2)

    xs = _sc_scatter_rows(x, idx3, G_CAP)
    zs = _expert_call(xs, expert_w1, expert_b1, expert_w2, expert_b2, proj_w,
                      be_arr, nb_arr, G_MAX)
    gath = _sc_gather(zs, destflat, chunk=32)
    out = _combine_call(gath, x, wt0, wt1, s1, s2)
    return out, gate_weights


# speedup vs baseline: 1.0899x; 1.0899x over previous
import functools

import jax
import jax.numpy as jnp
from jax import lax
from jax.experimental import pallas as pl
from jax.experimental.pallas import tpu as pltpu
from jax.experimental.pallas import tpu_sc as plsc

THRESH = 0.1
BM = 512
NC, NS = 2, 16
NW = NC * NS


def _route_body(x_ref, gw1_ref, gb1_ref, gw2_ref, gb2_ref, alpha_ref,
                gwout_ref, destcat_ref, wt0_ref, wt1_ref, s1_ref, s2_ref,
                benb_ref, g_max, g_cap):
    B = x_ref.shape[0]
    E = gw2_ref.shape[1]
    xb = x_ref[...]
    h = jnp.dot(xb, gw1_ref[...], preferred_element_type=jnp.float32)
    h = h + gb1_ref[...]
    h = h * jax.nn.sigmoid(h)
    logits = jnp.dot(h, gw2_ref[...], preferred_element_type=jnp.float32)
    logits = logits + gb2_ref[...]
    m = jnp.max(logits, axis=1, keepdims=True)
    p = jnp.exp(logits - m)
    gw = p / jnp.sum(p, axis=1, keepdims=True)
    gwout_ref[...] = gw

    lane = jax.lax.broadcasted_iota(jnp.int32, (B, E), 1)
    cols = []
    for ee in range(E):
        ge = gw[:, ee:ee + 1]
        gt = jnp.sum((gw > ge).astype(jnp.int32), axis=1, keepdims=True)
        eqb = jnp.sum(((gw == ge) & (lane < ee)).astype(jnp.int32),
                      axis=1, keepdims=True)
        fire = ((gt + eqb) < 2) & (ge > THRESH)
        cols.append(jnp.where(fire, ge, 0.0))
    w = jnp.concatenate(cols, axis=1)
    fire = w > 0.0
    fire_f = fire.astype(jnp.float32)

    RB = 512
    pos_blocks = []
    for rb in range(B // RB):
        ri = jax.lax.broadcasted_iota(jnp.int32, (RB, B), 0) + rb * RB
        ci = jax.lax.broadcasted_iota(jnp.int32, (RB, B), 1)
        lblk = (ci < ri).astype(jnp.float32)
        pos_blocks.append(jnp.dot(lblk, fire_f,
                                  preferred_element_type=jnp.float32))
    pos = jnp.concatenate(pos_blocks, axis=0)
    c_e = jnp.sum(fire_f, axis=0, keepdims=True)
    nb_e = jnp.floor_divide(c_e.astype(jnp.int32) + BM - 1, BM)
    ui = (jax.lax.broadcasted_iota(jnp.int32, (E, E), 0)
          <= jax.lax.broadcasted_iota(jnp.int32, (E, E), 1))
    nb_cum = jnp.dot(nb_e.astype(jnp.float32), ui.astype(jnp.float32),
                     preferred_element_type=jnp.float32)
    base = ((nb_cum - nb_e.astype(jnp.float32)) * BM)
    dest = (base + pos).astype(jnp.int32)

    big = jnp.int32(g_cap)
    dmin = jnp.min(jnp.where(fire, dest, big), axis=1, keepdims=True)
    dmax = jnp.max(jnp.where(fire, dest, -1), axis=1, keepdims=True)
    nf = jnp.sum(fire.astype(jnp.int32), axis=1, keepdims=True)
    w_min = jnp.sum(jnp.where(dest == dmin, w, 0.0), axis=1, keepdims=True)
    w_max = jnp.sum(jnp.where(dest == dmax, w, 0.0), axis=1, keepdims=True)
    toki = jax.lax.broadcasted_iota(jnp.int32, (B, 1), 0)
    dummy = (g_cap - BM) + (toki & (BM - 1))
    dest0 = jnp.where(nf >= 1, dmin, dummy)
    dest1 = jnp.where(nf >= 2, dmax, dummy)
    wt0 = w_min
    wt1 = jnp.where(nf >= 2, w_max, 0.0)
    destcat_ref[0:B, :] = dest0
    destcat_ref[B:2 * B, :] = dest1
    wt0_ref[...] = wt0
    wt1_ref[...] = wt1
    tw = wt0 + wt1
    fired = nf >= 1
    stw = jnp.where(fired, tw, 1.0)
    a = alpha_ref[0, 0]
    s1_ref[...] = a / stw
    s2_ref[...] = jnp.where(fired, 1.0 - a, 1.0)

    gi = jax.lax.broadcasted_iota(jnp.int32, (32, E), 0)
    nbc_i = nb_cum.astype(jnp.int32)
    be = jnp.sum((gi >= nbc_i).astype(jnp.int32), axis=1, keepdims=True)
    be = jnp.minimum(be, E - 1)
    rowi = jax.lax.broadcasted_iota(jnp.int32, (32, 1), 0)
    benb_ref[...] = jnp.where(rowi == g_max, nbc_i[:, E - 1:E], be)


def _route_call(x, gate_w1, gate_b1, gate_w2, gate_b2, alpha, g_max, g_cap):
    B, H = x.shape
    H2 = gate_w1.shape[1]
    E = gate_w2.shape[1]
    return pl.pallas_call(
        functools.partial(_route_body, g_max=g_max, g_cap=g_cap),
        in_specs=[
            pl.BlockSpec((B, H), lambda: (0, 0)),
            pl.BlockSpec((H, H2), lambda: (0, 0)),
            pl.BlockSpec((1, H2), lambda: (0, 0)),
            pl.BlockSpec((H2, E), lambda: (0, 0)),
            pl.BlockSpec((1, E), lambda: (0, 0)),
            pl.BlockSpec(memory_space=pltpu.SMEM),
        ],
        out_specs=[
            pl.BlockSpec((B, E), lambda: (0, 0)),
            pl.BlockSpec((2 * B, 1), lambda: (0, 0)),
            pl.BlockSpec((B, 1), lambda: (0, 0)),
            pl.BlockSpec((B, 1), lambda: (0, 0)),
            pl.BlockSpec((B, 1), lambda: (0, 0)),
            pl.BlockSpec((B, 1), lambda: (0, 0)),
            pl.BlockSpec((32, 1), lambda: (0, 0)),
        ],
        out_shape=[
            jax.ShapeDtypeStruct((B, E), jnp.float32),
            jax.ShapeDtypeStruct((2 * B, 1), jnp.int32),
            jax.ShapeDtypeStruct((B, 1), jnp.float32),
            jax.ShapeDtypeStruct((B, 1), jnp.float32),
            jax.ShapeDtypeStruct((B, 1), jnp.float32),
            jax.ShapeDtypeStruct((B, 1), jnp.float32),
            jax.ShapeDtypeStruct((32, 1), jnp.int32),
        ],
    )(x, gate_w1, gate_b1.reshape(1, H2), gate_w2, gate_b2.reshape(1, E),
      alpha)


def _sc_scatter_rows(x, idx3, n_out):
    B, D = x.shape
    tpw = B // NW
    mesh = plsc.VectorSubcoreMesh(core_axis_name="c", subcore_axis_name="s")

    @functools.partial(
        pl.kernel, mesh=mesh,
        out_type=jax.ShapeDtypeStruct((n_out, D), jnp.float32),
        scratch_types=[
            pltpu.VMEM((2, tpw), jnp.int32),
            pltpu.VMEM((tpw, D), jnp.float32),
            pltpu.SemaphoreType.DMA,
            pltpu.SemaphoreType.DMA,
        ],
    )
    def k(x_hbm, idx_hbm, out_hbm, idx_v, buf, sem0, sem1):
        wid = lax.axis_index("s") * NC + lax.axis_index("c")
        pltpu.sync_copy(idx_hbm.at[wid], idx_v)
        pltpu.sync_copy(x_hbm.at[pl.ds(wid * tpw, tpw)], buf)
        c0 = pltpu.async_copy(buf, out_hbm.at[idx_v.at[0]], sem0)
        c1 = pltpu.async_copy(buf, out_hbm.at[idx_v.at[1]], sem1)
        c0.wait()
        c1.wait()

    return k(x, idx3)


def _sc_gather(table, idx, chunk):
    n_rows = idx.shape[0]
    D = table.shape[1]
    rows_per_tile = n_rows // NW
    n_chunks = rows_per_tile // chunk
    mesh = plsc.VectorSubcoreMesh(core_axis_name="c", subcore_axis_name="s")

    @functools.partial(
        pl.kernel, mesh=mesh,
        out_type=jax.ShapeDtypeStruct((n_rows, D), jnp.float32),
        scratch_types=[
            pltpu.VMEM((rows_per_tile,), jnp.int32),
            pltpu.VMEM((chunk, D), jnp.float32),
            pltpu.VMEM((chunk, D), jnp.float32),
            pltpu.SemaphoreType.DMA,
            pltpu.SemaphoreType.DMA,
        ],
    )
    def k(table_hbm, idx_hbm, out_hbm, idx_v, buf0, buf1, sem_g, sem_s):
        wid = lax.axis_index("s") * NC + lax.axis_index("c")
        base = wid * rows_per_tile
        bufs = (buf0, buf1)
        pltpu.sync_copy(idx_hbm.at[pl.ds(base, rows_per_tile)], idx_v)
        gathers = []
        stores = []
        gathers.append(pltpu.async_copy(
            table_hbm.at[idx_v.at[pl.ds(0, chunk)]], bufs[0], sem_g))
        for c in range(n_chunks):
            gathers[c].wait()
            if c + 1 < n_chunks:
                if c >= 1:
                    stores[c - 1].wait()
                gathers.append(pltpu.async_copy(
                    table_hbm.at[idx_v.at[pl.ds((c + 1) * chunk, chunk)]],
                    bufs[(c + 1) % 2], sem_g))
            stores.append(pltpu.async_copy(
                bufs[c % 2], out_hbm.at[pl.ds(base + c * chunk, chunk)],
                sem_s))
        stores[n_chunks - 2].wait()
        stores[n_chunks - 1].wait()

    return k(table, idx)


def _expert_body(be_ref, nb_ref, xs_ref,
                 ew1_ref, eb1_ref, ew2_ref, eb2_ref, pw_ref, zs_ref):
    g = pl.program_id(0)

    @pl.when(g < nb_ref[0])
    def _compute():
        xb = xs_ref[...]
        h1 = jnp.dot(xb, ew1_ref[0], preferred_element_type=jnp.float32)
        h1 = h1 + eb1_ref[0]
        h1 = h1 * jax.nn.sigmoid(h1)
        eo = jnp.dot(h1, ew2_ref[0], preferred_element_type=jnp.float32)
        eo = eo + eb2_ref[0]
        zs_ref[...] = jnp.dot(eo, pw_ref[0], preferred_element_type=jnp.float32)

    @pl.when(g >= nb_ref[0])
    def _zero():
        zs_ref[...] = jnp.zeros_like(zs_ref)


def _expert_call(xs, expert_w1, expert_b1, expert_w2, expert_b2, proj_w,
                 block_expert, nb_arr, g_max):
    G_CAP, H = xs.shape
    E, _, F = expert_w1.shape
    grid_spec = pltpu.PrefetchScalarGridSpec(
        num_scalar_prefetch=2,
        grid=(g_max,),
        in_specs=[
            pl.BlockSpec((BM, H), lambda g, be, nb: (g, 0)),
            pl.BlockSpec((1, H, F), lambda g, be, nb: (be[g], 0, 0)),
            pl.BlockSpec((1, 1, F), lambda g, be, nb: (be[g], 0, 0)),
            pl.BlockSpec((1, F, H), lambda g, be, nb: (be[g], 0, 0)),
            pl.BlockSpec((1, 1, H), lambda g, be, nb: (be[g], 0, 0)),
            pl.BlockSpec((1, H, H), lambda g, be, nb: (be[g], 0, 0)),
        ],
        out_specs=pl.BlockSpec((BM, H), lambda g, be, nb: (g, 0)),
    )
    return pl.pallas_call(
        _expert_body,
        grid_spec=grid_spec,
        out_shape=jax.ShapeDtypeStruct((G_CAP, H), jnp.float32),
    )(block_expert, nb_arr, xs,
      expert_w1, expert_b1.reshape(E, 1, F), expert_w2,
      expert_b2.reshape(E, 1, H), proj_w)


def _combine_body(g0_ref, g1_ref, x_ref, wt0_ref, wt1_ref, s1_ref, s2_ref,
                  out_ref):
    acc = wt0_ref[...] * g0_ref[...] + wt1_ref[...] * g1_ref[...]
    out_ref[...] = s1_ref[...] * acc + s2_ref[...] * x_ref[...]


def _combine_call(gath, x, wt0, wt1, s1, s2):
    B, H = x.shape
    BMc = 512
    MB = B // BMc
    return pl.pallas_call(
        _combine_body,
        grid=(MB,),
        in_specs=[
            pl.BlockSpec((BMc, H), lambda mb: (mb, 0)),
            pl.BlockSpec((BMc, H), lambda mb, _MB=MB: (mb + _MB, 0)),
            pl.BlockSpec((BMc, H), lambda mb: (mb, 0)),
            pl.BlockSpec((BMc, 1), lambda mb: (mb, 0)),
            pl.BlockSpec((BMc, 1), lambda mb: (mb, 0)),
            pl.BlockSpec((BMc, 1), lambda mb: (mb, 0)),
            pl.BlockSpec((BMc, 1), lambda mb: (mb, 0)),
        ],
        out_specs=pl.BlockSpec((BMc, H), lambda mb: (mb, 0)),
        out_shape=jax.ShapeDtypeStruct((B, H), jnp.float32),
    )(gath, gath, x, wt0, wt1, s1, s2)


def kernel(x, gate_w1, gate_b1, gate_w2, gate_b2,
           expert_w1, expert_b1, expert_w2, expert_b2, proj_w, blend):
    B, H = x.shape
    E = gate_w2.shape[1]
    G_MAX = (2 * B) // BM + E + 1
    G_CAP = G_MAX * BM

    alpha = jax.nn.sigmoid(blend).reshape(1, 1).astype(jnp.float32)
    (gate_weights, destcat, wt0, wt1, s1, s2, benb) = _route_call(
        x, gate_w1, gate_b1, gate_w2, gate_b2, alpha, G_MAX, G_CAP)

    be_arr = benb[:G_MAX, 0]
    nb_arr = benb[G_MAX:G_MAX + 1, 0]
    destflat = destcat.reshape(2 * B)
    idx3 = destcat.reshape(2, NW, B // NW).transpose(1, 0, 2)

    xs = _sc_scatter_rows(x, idx3, G_CAP)
    zs = _expert_call(xs, expert_w1, expert_b1, expert_w2, expert_b2, proj_w,
                      be_arr, nb_arr, G_MAX)
    gath = _sc_gather(zs, destflat, chunk=32)
    out = _combine_call(gath, x, wt0, wt1, s1, s2)
    return out, gate_weights


# speedup vs baseline: 1.0907x; 1.0007x over previous
import functools

import jax
import jax.numpy as jnp
from jax import lax
from jax.experimental import pallas as pl
from jax.experimental.pallas import tpu as pltpu
from jax.experimental.pallas import tpu_sc as plsc

THRESH = 0.1
BM = 512
NC, NS = 2, 16
NW = NC * NS


def _route_body(x_ref, gw1_ref, gb1_ref, gw2_ref, gb2_ref, alpha_ref,
                gwout_ref, destcat_ref, wt0_ref, wt1_ref, s1_ref, s2_ref,
                benb_ref, g_max, g_cap):
    B = x_ref.shape[0]
    E = gw2_ref.shape[1]
    xb = x_ref[...]
    h = jnp.dot(xb, gw1_ref[...], preferred_element_type=jnp.float32)
    h = h + gb1_ref[...]
    h = h * jax.nn.sigmoid(h)
    logits = jnp.dot(h, gw2_ref[...], preferred_element_type=jnp.float32)
    logits = logits + gb2_ref[...]
    m = jnp.max(logits, axis=1, keepdims=True)
    p = jnp.exp(logits - m)
    gw = p / jnp.sum(p, axis=1, keepdims=True)
    gwout_ref[...] = gw

    lane = jax.lax.broadcasted_iota(jnp.int32, (B, E), 1)
    cols = []
    for ee in range(E):
        ge = gw[:, ee:ee + 1]
        gt = jnp.sum((gw > ge).astype(jnp.int32), axis=1, keepdims=True)
        eqb = jnp.sum(((gw == ge) & (lane < ee)).astype(jnp.int32),
                      axis=1, keepdims=True)
        fire = ((gt + eqb) < 2) & (ge > THRESH)
        cols.append(jnp.where(fire, ge, 0.0))
    w = jnp.concatenate(cols, axis=1)
    fire = w > 0.0
    fire_f = fire.astype(jnp.float32)

    RB = 512
    pos_blocks = []
    for rb in range(B // RB):
        ri = jax.lax.broadcasted_iota(jnp.int32, (RB, B), 0) + rb * RB
        ci = jax.lax.broadcasted_iota(jnp.int32, (RB, B), 1)
        lblk = (ci < ri).astype(jnp.float32)
        pos_blocks.append(jnp.dot(lblk, fire_f,
                                  preferred_element_type=jnp.float32))
    pos = jnp.concatenate(pos_blocks, axis=0)
    c_e = jnp.sum(fire_f, axis=0, keepdims=True)
    nb_e = jnp.floor_divide(c_e.astype(jnp.int32) + BM - 1, BM)
    ui = (jax.lax.broadcasted_iota(jnp.int32, (E, E), 0)
          <= jax.lax.broadcasted_iota(jnp.int32, (E, E), 1))
    nb_cum = jnp.dot(nb_e.astype(jnp.float32), ui.astype(jnp.float32),
                     preferred_element_type=jnp.float32)
    base = ((nb_cum - nb_e.astype(jnp.float32)) * BM)
    dest = (base + pos).astype(jnp.int32)

    big = jnp.int32(g_cap)
    dmin = jnp.min(jnp.where(fire, dest, big), axis=1, keepdims=True)
    dmax = jnp.max(jnp.where(fire, dest, -1), axis=1, keepdims=True)
    nf = jnp.sum(fire.astype(jnp.int32), axis=1, keepdims=True)
    w_min = jnp.sum(jnp.where(dest == dmin, w, 0.0), axis=1, keepdims=True)
    w_max = jnp.sum(jnp.where(dest == dmax, w, 0.0), axis=1, keepdims=True)
    toki = jax.lax.broadcasted_iota(jnp.int32, (B, 1), 0)
    dummy = (g_cap - BM) + (toki & (BM - 1))
    dest0 = jnp.where(nf >= 1, dmin, dummy)
    dest1 = jnp.where(nf >= 2, dmax, dummy)
    wt0 = w_min
    wt1 = jnp.where(nf >= 2, w_max, 0.0)
    destcat_ref[0:B, :] = dest0
    destcat_ref[B:2 * B, :] = dest1
    wt0_ref[...] = wt0
    wt1_ref[...] = wt1
    tw = wt0 + wt1
    fired = nf >= 1
    stw = jnp.where(fired, tw, 1.0)
    a = alpha_ref[0, 0]
    s1_ref[...] = a / stw
    s2_ref[...] = jnp.where(fired, 1.0 - a, 1.0)

    gi = jax.lax.broadcasted_iota(jnp.int32, (32, E), 0)
    nbc_i = nb_cum.astype(jnp.int32)
    be = jnp.sum((gi >= nbc_i).astype(jnp.int32), axis=1, keepdims=True)
    be = jnp.minimum(be, E - 1)
    rowi = jax.lax.broadcasted_iota(jnp.int32, (32, 1), 0)
    benb_ref[...] = jnp.where(rowi == g_max, nbc_i[:, E - 1:E], be)


def _route_call(x, gate_w1, gate_b1, gate_w2, gate_b2, alpha, g_max, g_cap):
    B, H = x.shape
    H2 = gate_w1.shape[1]
    E = gate_w2.shape[1]
    return pl.pallas_call(
        functools.partial(_route_body, g_max=g_max, g_cap=g_cap),
        in_specs=[
            pl.BlockSpec((B, H), lambda: (0, 0)),
            pl.BlockSpec((H, H2), lambda: (0, 0)),
            pl.BlockSpec((1, H2), lambda: (0, 0)),
            pl.BlockSpec((H2, E), lambda: (0, 0)),
            pl.BlockSpec((1, E), lambda: (0, 0)),
            pl.BlockSpec(memory_space=pltpu.SMEM),
        ],
        out_specs=[
            pl.BlockSpec((B, E), lambda: (0, 0)),
            pl.BlockSpec((2 * B, 1), lambda: (0, 0)),
            pl.BlockSpec((B, 1), lambda: (0, 0)),
            pl.BlockSpec((B, 1), lambda: (0, 0)),
            pl.BlockSpec((B, 1), lambda: (0, 0)),
            pl.BlockSpec((B, 1), lambda: (0, 0)),
            pl.BlockSpec((32, 1), lambda: (0, 0)),
        ],
        out_shape=[
            jax.ShapeDtypeStruct((B, E), jnp.float32),
            jax.ShapeDtypeStruct((2 * B, 1), jnp.int32),
            jax.ShapeDtypeStruct((B, 1), jnp.float32),
            jax.ShapeDtypeStruct((B, 1), jnp.float32),
            jax.ShapeDtypeStruct((B, 1), jnp.float32),
            jax.ShapeDtypeStruct((B, 1), jnp.float32),
            jax.ShapeDtypeStruct((32, 1), jnp.int32),
        ],
    )(x, gate_w1, gate_b1.reshape(1, H2), gate_w2, gate_b2.reshape(1, E),
      alpha)


def _sc_scatter_rows(x, idx3, n_out):
    B, D = x.shape
    tpw = B // NW
    mesh = plsc.VectorSubcoreMesh(core_axis_name="c", subcore_axis_name="s")

    @functools.partial(
        pl.kernel, mesh=mesh,
        out_type=jax.ShapeDtypeStruct((n_out, D), jnp.float32),
        scratch_types=[
            pltpu.VMEM((2, tpw), jnp.int32),
            pltpu.VMEM((tpw, D), jnp.float32),
            pltpu.SemaphoreType.DMA,
            pltpu.SemaphoreType.DMA,
        ],
    )
    def k(x_hbm, idx_hbm, out_hbm, idx_v, buf, sem0, sem1):
        wid = lax.axis_index("s") * NC + lax.axis_index("c")
        pltpu.sync_copy(idx_hbm.at[wid], idx_v)
        pltpu.sync_copy(x_hbm.at[pl.ds(wid * tpw, tpw)], buf)
        c0 = pltpu.async_copy(buf, out_hbm.at[idx_v.at[0]], sem0)
        c1 = pltpu.async_copy(buf, out_hbm.at[idx_v.at[1]], sem1)
        c0.wait()
        c1.wait()

    return k(x, idx3)


def _sc_gather(table, idx, chunk):
    n_rows = idx.shape[0]
    D = table.shape[1]
    rows_per_tile = n_rows // NW
    n_chunks = rows_per_tile // chunk
    mesh = plsc.VectorSubcoreMesh(core_axis_name="c", subcore_axis_name="s")

    @functools.partial(
        pl.kernel, mesh=mesh,
        out_type=jax.ShapeDtypeStruct((n_rows, D), jnp.float32),
        scratch_types=[
            pltpu.VMEM((rows_per_tile,), jnp.int32),
            pltpu.VMEM((chunk, D), jnp.float32),
            pltpu.VMEM((chunk, D), jnp.float32),
            pltpu.SemaphoreType.DMA,
            pltpu.SemaphoreType.DMA,
        ],
    )
    def k(table_hbm, idx_hbm, out_hbm, idx_v, buf0, buf1, sem_g, sem_s):
        wid = lax.axis_index("s") * NC + lax.axis_index("c")
        base = wid * rows_per_tile
        bufs = (buf0, buf1)
        pltpu.sync_copy(idx_hbm.at[pl.ds(base, rows_per_tile)], idx_v)
        gathers = []
        stores = []
        gathers.append(pltpu.async_copy(
            table_hbm.at[idx_v.at[pl.ds(0, chunk)]], bufs[0], sem_g))
        for c in range(n_chunks):
            gathers[c].wait()
            if c + 1 < n_chunks:
                if c >= 1:
                    stores[c - 1].wait()
                gathers.append(pltpu.async_copy(
                    table_hbm.at[idx_v.at[pl.ds((c + 1) * chunk, chunk)]],
                    bufs[(c + 1) % 2], sem_g))
            stores.append(pltpu.async_copy(
                bufs[c % 2], out_hbm.at[pl.ds(base + c * chunk, chunk)],
                sem_s))
        stores[n_chunks - 2].wait()
        stores[n_chunks - 1].wait()

    return k(table, idx)


def _expert_body(be_ref, nb_ref, xs_ref,
                 ew1_ref, eb1_ref, ew2_ref, eb2_ref, pw_ref, zs_ref):
    g = pl.program_id(0)

    @pl.when(g < nb_ref[0])
    def _compute():
        bf = jnp.bfloat16
        xb = xs_ref[...].astype(bf)
        h1 = jnp.dot(xb, ew1_ref[0].astype(bf),
                     preferred_element_type=jnp.float32)
        h1 = h1 + eb1_ref[0]
        h1 = h1 * jax.nn.sigmoid(h1)
        eo = jnp.dot(h1.astype(bf), ew2_ref[0].astype(bf),
                     preferred_element_type=jnp.float32)
        eo = eo + eb2_ref[0]
        zs_ref[...] = jnp.dot(eo.astype(bf), pw_ref[0].astype(bf),
                              preferred_element_type=jnp.float32)

    @pl.when(g >= nb_ref[0])
    def _zero():
        zs_ref[...] = jnp.zeros_like(zs_ref)


def _expert_call(xs, expert_w1, expert_b1, expert_w2, expert_b2, proj_w,
                 block_expert, nb_arr, g_max):
    G_CAP, H = xs.shape
    E, _, F = expert_w1.shape
    grid_spec = pltpu.PrefetchScalarGridSpec(
        num_scalar_prefetch=2,
        grid=(g_max,),
        in_specs=[
            pl.BlockSpec((BM, H), lambda g, be, nb: (g, 0)),
            pl.BlockSpec((1, H, F), lambda g, be, nb: (be[g], 0, 0)),
            pl.BlockSpec((1, 1, F), lambda g, be, nb: (be[g], 0, 0)),
            pl.BlockSpec((1, F, H), lambda g, be, nb: (be[g], 0, 0)),
            pl.BlockSpec((1, 1, H), lambda g, be, nb: (be[g], 0, 0)),
            pl.BlockSpec((1, H, H), lambda g, be, nb: (be[g], 0, 0)),
        ],
        out_specs=pl.BlockSpec((BM, H), lambda g, be, nb: (g, 0)),
    )
    return pl.pallas_call(
        _expert_body,
        grid_spec=grid_spec,
        out_shape=jax.ShapeDtypeStruct((G_CAP, H), jnp.float32),
    )(block_expert, nb_arr, xs,
      expert_w1, expert_b1.reshape(E, 1, F), expert_w2,
      expert_b2.reshape(E, 1, H), proj_w)


def _combine_body(g0_ref, g1_ref, x_ref, wt0_ref, wt1_ref, s1_ref, s2_ref,
                  out_ref):
    acc = wt0_ref[...] * g0_ref[...] + wt1_ref[...] * g1_ref[...]
    out_ref[...] = s1_ref[...] * acc + s2_ref[...] * x_ref[...]


def _combine_call(gath, x, wt0, wt1, s1, s2):
    B, H = x.shape
    BMc = 512
    MB = B // BMc
    return pl.pallas_call(
        _combine_body,
        grid=(MB,),
        in_specs=[
            pl.BlockSpec((BMc, H), lambda mb: (mb, 0)),
            pl.BlockSpec((BMc, H), lambda mb, _MB=MB: (mb + _MB, 0)),
            pl.BlockSpec((BMc, H), lambda mb: (mb, 0)),
            pl.BlockSpec((BMc, 1), lambda mb: (mb, 0)),
            pl.BlockSpec((BMc, 1), lambda mb: (mb, 0)),
            pl.BlockSpec((BMc, 1), lambda mb: (mb, 0)),
            pl.BlockSpec((BMc, 1), lambda mb: (mb, 0)),
        ],
        out_specs=pl.BlockSpec((BMc, H), lambda mb: (mb, 0)),
        out_shape=jax.ShapeDtypeStruct((B, H), jnp.float32),
    )(gath, gath, x, wt0, wt1, s1, s2)


def kernel(x, gate_w1, gate_b1, gate_w2, gate_b2,
           expert_w1, expert_b1, expert_w2, expert_b2, proj_w, blend):
    B, H = x.shape
    E = gate_w2.shape[1]
    G_MAX = (2 * B) // BM + E + 1
    G_CAP = G_MAX * BM

    alpha = jax.nn.sigmoid(blend).reshape(1, 1).astype(jnp.float32)
    (gate_weights, destcat, wt0, wt1, s1, s2, benb) = _route_call(
        x, gate_w1, gate_b1, gate_w2, gate_b2, alpha, G_MAX, G_CAP)

    be_arr = benb[:G_MAX, 0]
    nb_arr = benb[G_MAX:G_MAX + 1, 0]
    destflat = destcat.reshape(2 * B)
    idx3 = destcat.reshape(2, NW, B // NW).transpose(1, 0, 2)

    xs = _sc_scatter_rows(x, idx3, G_CAP)
    zs = _expert_call(xs, expert_w1, expert_b1, expert_w2, expert_b2, proj_w,
                      be_arr, nb_arr, G_MAX)
    gath = _sc_gather(zs, destflat, chunk=32)
    out = _combine_call(gath, x, wt0, wt1, s1, s2)
    return out, gate_weights


# speedup vs baseline: 1.1264x; 1.0327x over previous
import functools

import jax
import jax.numpy as jnp
from jax import lax
from jax.experimental import pallas as pl
from jax.experimental.pallas import tpu as pltpu
from jax.experimental.pallas import tpu_sc as plsc

THRESH = 0.1
BM = 512
NC, NS = 2, 16
NW = NC * NS


def _route_body(x_ref, gw1_ref, gb1_ref, gw2_ref, gb2_ref, alpha_ref,
                gwout_ref, destcat_ref, idx3_ref, wt0_ref, wt1_ref,
                s1_ref, s2_ref, benb_ref, g_max, g_cap):
    B = x_ref.shape[0]
    E = gw2_ref.shape[1]
    xb = x_ref[...]
    h = jnp.dot(xb, gw1_ref[...], preferred_element_type=jnp.float32)
    h = h + gb1_ref[...]
    h = h * jax.nn.sigmoid(h)
    logits = jnp.dot(h, gw2_ref[...], preferred_element_type=jnp.float32)
    logits = logits + gb2_ref[...]
    m = jnp.max(logits, axis=1, keepdims=True)
    p = jnp.exp(logits - m)
    gw = p / jnp.sum(p, axis=1, keepdims=True)
    gwout_ref[...] = gw

    lane = jax.lax.broadcasted_iota(jnp.int32, (B, E), 1)
    cols = []
    for ee in range(E):
        ge = gw[:, ee:ee + 1]
        gt = jnp.sum((gw > ge).astype(jnp.int32), axis=1, keepdims=True)
        eqb = jnp.sum(((gw == ge) & (lane < ee)).astype(jnp.int32),
                      axis=1, keepdims=True)
        fire = ((gt + eqb) < 2) & (ge > THRESH)
        cols.append(jnp.where(fire, ge, 0.0))
    w = jnp.concatenate(cols, axis=1)
    fire = w > 0.0
    fire_f = fire.astype(jnp.float32)

    RB = 512
    pos_blocks = []
    for rb in range(B // RB):
        ri = jax.lax.broadcasted_iota(jnp.int32, (RB, B), 0) + rb * RB
        ci = jax.lax.broadcasted_iota(jnp.int32, (RB, B), 1)
        lblk = (ci < ri).astype(jnp.float32)
        pos_blocks.append(jnp.dot(lblk, fire_f,
                                  preferred_element_type=jnp.float32))
    pos = jnp.concatenate(pos_blocks, axis=0)
    c_e = jnp.sum(fire_f, axis=0, keepdims=True)
    nb_e = jnp.floor_divide(c_e.astype(jnp.int32) + BM - 1, BM)
    ui = (jax.lax.broadcasted_iota(jnp.int32, (E, E), 0)
          <= jax.lax.broadcasted_iota(jnp.int32, (E, E), 1))
    nb_cum = jnp.dot(nb_e.astype(jnp.float32), ui.astype(jnp.float32),
                     preferred_element_type=jnp.float32)
    base = ((nb_cum - nb_e.astype(jnp.float32)) * BM)
    dest = (base + pos).astype(jnp.int32)

    big = jnp.int32(g_cap)
    dmin = jnp.min(jnp.where(fire, dest, big), axis=1, keepdims=True)
    dmax = jnp.max(jnp.where(fire, dest, -1), axis=1, keepdims=True)
    nf = jnp.sum(fire.astype(jnp.int32), axis=1, keepdims=True)
    w_min = jnp.sum(jnp.where(dest == dmin, w, 0.0), axis=1, keepdims=True)
    w_max = jnp.sum(jnp.where(dest == dmax, w, 0.0), axis=1, keepdims=True)
    toki = jax.lax.broadcasted_iota(jnp.int32, (B, 1), 0)
    dummy = (g_cap - BM) + (toki & (BM - 1))
    dest0 = jnp.where(nf >= 1, dmin, dummy)
    dest1 = jnp.where(nf >= 2, dmax, dummy)
    wt0 = w_min
    wt1 = jnp.where(nf >= 2, w_max, 0.0)
    destcat_ref[0:B, :] = dest0
    destcat_ref[B:2 * B, :] = dest1
    tpw = B // NW
    idx3_ref[:, 0, :] = jnp.reshape(dest0, (NW, tpw))
    idx3_ref[:, 1, :] = jnp.reshape(dest1, (NW, tpw))
    wt0_ref[...] = wt0
    wt1_ref[...] = wt1
    tw = wt0 + wt1
    fired = nf >= 1
    stw = jnp.where(fired, tw, 1.0)
    a = alpha_ref[0, 0]
    s1_ref[...] = a / stw
    s2_ref[...] = jnp.where(fired, 1.0 - a, 1.0)

    gi = jax.lax.broadcasted_iota(jnp.int32, (32, E), 0)
    nbc_i = nb_cum.astype(jnp.int32)
    be = jnp.sum((gi >= nbc_i).astype(jnp.int32), axis=1, keepdims=True)
    be = jnp.minimum(be, E - 1)
    rowi = jax.lax.broadcasted_iota(jnp.int32, (32, 1), 0)
    benb_ref[...] = jnp.where(rowi == g_max, nbc_i[:, E - 1:E], be)


def _route_call(x, gate_w1, gate_b1, gate_w2, gate_b2, alpha, g_max, g_cap):
    B, H = x.shape
    H2 = gate_w1.shape[1]
    E = gate_w2.shape[1]
    return pl.pallas_call(
        functools.partial(_route_body, g_max=g_max, g_cap=g_cap),
        in_specs=[
            pl.BlockSpec((B, H), lambda: (0, 0)),
            pl.BlockSpec((H, H2), lambda: (0, 0)),
            pl.BlockSpec((1, H2), lambda: (0, 0)),
            pl.BlockSpec((H2, E), lambda: (0, 0)),
            pl.BlockSpec((1, E), lambda: (0, 0)),
            pl.BlockSpec(memory_space=pltpu.SMEM),
        ],
        out_specs=[
            pl.BlockSpec((B, E), lambda: (0, 0)),
            pl.BlockSpec((2 * B, 1), lambda: (0, 0)),
            pl.BlockSpec((NW, 2, B // NW), lambda: (0, 0, 0)),
            pl.BlockSpec((B, 1), lambda: (0, 0)),
            pl.BlockSpec((B, 1), lambda: (0, 0)),
            pl.BlockSpec((B, 1), lambda: (0, 0)),
            pl.BlockSpec((B, 1), lambda: (0, 0)),
            pl.BlockSpec((32, 1), lambda: (0, 0)),
        ],
        out_shape=[
            jax.ShapeDtypeStruct((B, E), jnp.float32),
            jax.ShapeDtypeStruct((2 * B, 1), jnp.int32),
            jax.ShapeDtypeStruct((NW, 2, B // NW), jnp.int32),
            jax.ShapeDtypeStruct((B, 1), jnp.float32),
            jax.ShapeDtypeStruct((B, 1), jnp.float32),
            jax.ShapeDtypeStruct((B, 1), jnp.float32),
            jax.ShapeDtypeStruct((B, 1), jnp.float32),
            jax.ShapeDtypeStruct((32, 1), jnp.int32),
        ],
    )(x, gate_w1, gate_b1.reshape(1, H2), gate_w2, gate_b2.reshape(1, E),
      alpha)


def _sc_scatter_rows(x, idx3, n_out):
    B, D = x.shape
    tpw = B // NW
    mesh = plsc.VectorSubcoreMesh(core_axis_name="c", subcore_axis_name="s")

    @functools.partial(
        pl.kernel, mesh=mesh,
        out_type=jax.ShapeDtypeStruct((n_out, D), jnp.float32),
        scratch_types=[
            pltpu.VMEM((2, tpw), jnp.int32),
            pltpu.VMEM((tpw, D), jnp.float32),
            pltpu.SemaphoreType.DMA,
            pltpu.SemaphoreType.DMA,
        ],
    )
    def k(x_hbm, idx_hbm, out_hbm, idx_v, buf, sem0, sem1):
        wid = lax.axis_index("s") * NC + lax.axis_index("c")
        pltpu.sync_copy(idx_hbm.at[wid], idx_v)
        pltpu.sync_copy(x_hbm.at[pl.ds(wid * tpw, tpw)], buf)
        c0 = pltpu.async_copy(buf, out_hbm.at[idx_v.at[0]], sem0)
        c1 = pltpu.async_copy(buf, out_hbm.at[idx_v.at[1]], sem1)
        c0.wait()
        c1.wait()

    return k(x, idx3)


def _sc_gather(table, idx, chunk):
    n_rows = idx.shape[0]
    D = table.shape[1]
    rows_per_tile = n_rows // NW
    n_chunks = rows_per_tile // chunk
    mesh = plsc.VectorSubcoreMesh(core_axis_name="c", subcore_axis_name="s")

    @functools.partial(
        pl.kernel, mesh=mesh,
        out_type=jax.ShapeDtypeStruct((n_rows, D), jnp.float32),
        scratch_types=[
            pltpu.VMEM((rows_per_tile,), jnp.int32),
            pltpu.VMEM((chunk, D), jnp.float32),
            pltpu.VMEM((chunk, D), jnp.float32),
            pltpu.SemaphoreType.DMA,
            pltpu.SemaphoreType.DMA,
        ],
    )
    def k(table_hbm, idx_hbm, out_hbm, idx_v, buf0, buf1, sem_g, sem_s):
        wid = lax.axis_index("s") * NC + lax.axis_index("c")
        base = wid * rows_per_tile
        bufs = (buf0, buf1)
        pltpu.sync_copy(idx_hbm.at[pl.ds(base, rows_per_tile)], idx_v)
        gathers = []
        stores = []
        gathers.append(pltpu.async_copy(
            table_hbm.at[idx_v.at[pl.ds(0, chunk)]], bufs[0], sem_g))
        for c in range(n_chunks):
            gathers[c].wait()
            if c + 1 < n_chunks:
                if c >= 1:
                    stores[c - 1].wait()
                gathers.append(pltpu.async_copy(
                    table_hbm.at[idx_v.at[pl.ds((c + 1) * chunk, chunk)]],
                    bufs[(c + 1) % 2], sem_g))
            stores.append(pltpu.async_copy(
                bufs[c % 2], out_hbm.at[pl.ds(base + c * chunk, chunk)],
                sem_s))
        stores[n_chunks - 2].wait()
        stores[n_chunks - 1].wait()

    return k(table, idx)


def _expert_body(be_ref, nb_ref, xs_ref,
                 ew1_ref, eb1_ref, ew2_ref, eb2_ref, pw_ref, zs_ref):
    g = pl.program_id(0)

    @pl.when(g < nb_ref[0])
    def _compute():
        xb = xs_ref[...]
        h1 = jnp.dot(xb, ew1_ref[0], preferred_element_type=jnp.float32)
        h1 = h1 + eb1_ref[0]
        h1 = h1 * jax.nn.sigmoid(h1)
        eo = jnp.dot(h1, ew2_ref[0], preferred_element_type=jnp.float32)
        eo = eo + eb2_ref[0]
        zs_ref[...] = jnp.dot(eo, pw_ref[0], preferred_element_type=jnp.float32)

    @pl.when(g >= nb_ref[0])
    def _zero():
        zs_ref[...] = jnp.zeros_like(zs_ref)


def _expert_call(xs, expert_w1, expert_b1, expert_w2, expert_b2, proj_w,
                 block_expert, nb_arr, g_max):
    G_CAP, H = xs.shape
    E, _, F = expert_w1.shape
    grid_spec = pltpu.PrefetchScalarGridSpec(
        num_scalar_prefetch=2,
        grid=(g_max,),
        in_specs=[
            pl.BlockSpec((BM, H), lambda g, be, nb: (g, 0)),
            pl.BlockSpec((1, H, F), lambda g, be, nb: (be[g], 0, 0)),
            pl.BlockSpec((1, 1, F), lambda g, be, nb: (be[g], 0, 0)),
            pl.BlockSpec((1, F, H), lambda g, be, nb: (be[g], 0, 0)),
            pl.BlockSpec((1, 1, H), lambda g, be, nb: (be[g], 0, 0)),
            pl.BlockSpec((1, H, H), lambda g, be, nb: (be[g], 0, 0)),
        ],
        out_specs=pl.BlockSpec(
            (BM, H),
            lambda g, be, nb: (jnp.where(g < nb[0], g, g_max - 1), 0)),
    )
    return pl.pallas_call(
        _expert_body,
        grid_spec=grid_spec,
        out_shape=jax.ShapeDtypeStruct((G_CAP, H), jnp.float32),
    )(block_expert, nb_arr, xs,
      expert_w1, expert_b1.reshape(E, 1, F), expert_w2,
      expert_b2.reshape(E, 1, H), proj_w)


def _combine_body(g0_ref, g1_ref, x_ref, wt0_ref, wt1_ref, s1_ref, s2_ref,
                  out_ref):
    acc = wt0_ref[...] * g0_ref[...] + wt1_ref[...] * g1_ref[...]
    out_ref[...] = s1_ref[...] * acc + s2_ref[...] * x_ref[...]


def _combine_call(gath, x, wt0, wt1, s1, s2):
    B, H = x.shape
    BMc = 512
    MB = B // BMc
    return pl.pallas_call(
        _combine_body,
        grid=(MB,),
        in_specs=[
            pl.BlockSpec((BMc, H), lambda mb: (mb, 0)),
            pl.BlockSpec((BMc, H), lambda mb, _MB=MB: (mb + _MB, 0)),
            pl.BlockSpec((BMc, H), lambda mb: (mb, 0)),
            pl.BlockSpec((BMc, 1), lambda mb: (mb, 0)),
            pl.BlockSpec((BMc, 1), lambda mb: (mb, 0)),
            pl.BlockSpec((BMc, 1), lambda mb: (mb, 0)),
            pl.BlockSpec((BMc, 1), lambda mb: (mb, 0)),
        ],
        out_specs=pl.BlockSpec((BMc, H), lambda mb: (mb, 0)),
        out_shape=jax.ShapeDtypeStruct((B, H), jnp.float32),
    )(gath, gath, x, wt0, wt1, s1, s2)


def kernel(x, gate_w1, gate_b1, gate_w2, gate_b2,
           expert_w1, expert_b1, expert_w2, expert_b2, proj_w, blend):
    B, H = x.shape
    E = gate_w2.shape[1]
    G_MAX = (2 * B) // BM + E + 1
    G_CAP = G_MAX * BM

    alpha = jax.nn.sigmoid(blend).reshape(1, 1).astype(jnp.float32)
    (gate_weights, destcat, idx3, wt0, wt1, s1, s2, benb) = _route_call(
        x, gate_w1, gate_b1, gate_w2, gate_b2, alpha, G_MAX, G_CAP)

    be_arr = benb[:G_MAX, 0]
    nb_arr = benb[G_MAX:G_MAX + 1, 0]
    destflat = destcat.reshape(2 * B)

    xs = _sc_scatter_rows(x, idx3, G_CAP)
    zs = _expert_call(xs, expert_w1, expert_b1, expert_w2, expert_b2, proj_w,
                      be_arr, nb_arr, G_MAX)
    gath = _sc_gather(zs, destflat, chunk=32)
    out = _combine_call(gath, x, wt0, wt1, s1, s2)
    return out, gate_weights


# speedup vs baseline: 1.1312x; 1.0043x over previous
import functools

import jax
import jax.numpy as jnp
from jax import lax
from jax.experimental import pallas as pl
from jax.experimental.pallas import tpu as pltpu
from jax.experimental.pallas import tpu_sc as plsc

THRESH = 0.1
BM = 512
NC, NS = 2, 16
NW = NC * NS


def _route_body(x_ref, gw1_ref, gb1_ref, gw2_ref, gb2_ref, alpha_ref,
                gwout_ref, destcat_ref, idx3_ref, wt0_ref, wt1_ref,
                s1_ref, s2_ref, benb_ref, g_max, g_cap):
    B = x_ref.shape[0]
    E = gw2_ref.shape[1]
    xb = x_ref[...]
    h = jnp.dot(xb, gw1_ref[...], preferred_element_type=jnp.float32)
    h = h + gb1_ref[...]
    h = h * jax.nn.sigmoid(h)
    logits = jnp.dot(h, gw2_ref[...], preferred_element_type=jnp.float32)
    logits = logits + gb2_ref[...]
    m = jnp.max(logits, axis=1, keepdims=True)
    p = jnp.exp(logits - m)
    gw = p / jnp.sum(p, axis=1, keepdims=True)
    gwout_ref[...] = gw

    lane = jax.lax.broadcasted_iota(jnp.int32, (B, E), 1)
    cols = []
    for ee in range(E):
        ge = gw[:, ee:ee + 1]
        gt = jnp.sum((gw > ge).astype(jnp.int32), axis=1, keepdims=True)
        eqb = jnp.sum(((gw == ge) & (lane < ee)).astype(jnp.int32),
                      axis=1, keepdims=True)
        fire = ((gt + eqb) < 2) & (ge > THRESH)
        cols.append(jnp.where(fire, ge, 0.0))
    w = jnp.concatenate(cols, axis=1)
    fire = w > 0.0
    fire_f = fire.astype(jnp.float32)

    RB = 512
    pos_blocks = []
    for rb in range(B // RB):
        ri = jax.lax.broadcasted_iota(jnp.int32, (RB, B), 0) + rb * RB
        ci = jax.lax.broadcasted_iota(jnp.int32, (RB, B), 1)
        lblk = (ci < ri).astype(jnp.float32)
        pos_blocks.append(jnp.dot(lblk, fire_f,
                                  preferred_element_type=jnp.float32))
    pos = jnp.concatenate(pos_blocks, axis=0)
    c_e = jnp.sum(fire_f, axis=0, keepdims=True)
    nb_e = jnp.floor_divide(c_e.astype(jnp.int32) + BM - 1, BM)
    ui = (jax.lax.broadcasted_iota(jnp.int32, (E, E), 0)
          <= jax.lax.broadcasted_iota(jnp.int32, (E, E), 1))
    nb_cum = jnp.dot(nb_e.astype(jnp.float32), ui.astype(jnp.float32),
                     preferred_element_type=jnp.float32)
    base = ((nb_cum - nb_e.astype(jnp.float32)) * BM)
    dest = (base + pos).astype(jnp.int32)

    big = jnp.int32(g_cap)
    dmin = jnp.min(jnp.where(fire, dest, big), axis=1, keepdims=True)
    dmax = jnp.max(jnp.where(fire, dest, -1), axis=1, keepdims=True)
    nf = jnp.sum(fire.astype(jnp.int32), axis=1, keepdims=True)
    w_min = jnp.sum(jnp.where(dest == dmin, w, 0.0), axis=1, keepdims=True)
    w_max = jnp.sum(jnp.where(dest == dmax, w, 0.0), axis=1, keepdims=True)
    toki = jax.lax.broadcasted_iota(jnp.int32, (B, 1), 0)
    dummy = (g_cap - BM) + (toki & (BM - 1))
    dest0 = jnp.where(nf >= 1, dmin, dummy)
    dest1 = jnp.where(nf >= 2, dmax, dummy)
    wt0 = w_min
    wt1 = jnp.where(nf >= 2, w_max, 0.0)
    destcat_ref[0:B, :] = dest0
    destcat_ref[B:2 * B, :] = dest1
    tpw = B // NW
    idx3_ref[:, 0, :] = jnp.reshape(dest0, (NW, tpw))
    idx3_ref[:, 1, :] = jnp.reshape(dest1, (NW, tpw))
    wt0_ref[...] = wt0
    wt1_ref[...] = wt1
    tw = wt0 + wt1
    fired = nf >= 1
    stw = jnp.where(fired, tw, 1.0)
    a = alpha_ref[0, 0]
    s1_ref[...] = a / stw
    s2_ref[...] = jnp.where(fired, 1.0 - a, 1.0)

    gi = jax.lax.broadcasted_iota(jnp.int32, (32, E), 0)
    nbc_i = nb_cum.astype(jnp.int32)
    be = jnp.sum((gi >= nbc_i).astype(jnp.int32), axis=1, keepdims=True)
    be = jnp.minimum(be, E - 1)
    rowi = jax.lax.broadcasted_iota(jnp.int32, (32, 1), 0)
    benb_ref[...] = jnp.where(rowi == g_max, nbc_i[:, E - 1:E], be)


def _route_call(x, gate_w1, gate_b1, gate_w2, gate_b2, alpha, g_max, g_cap):
    B, H = x.shape
    H2 = gate_w1.shape[1]
    E = gate_w2.shape[1]
    return pl.pallas_call(
        functools.partial(_route_body, g_max=g_max, g_cap=g_cap),
        in_specs=[
            pl.BlockSpec((B, H), lambda: (0, 0)),
            pl.BlockSpec((H, H2), lambda: (0, 0)),
            pl.BlockSpec((1, H2), lambda: (0, 0)),
            pl.BlockSpec((H2, E), lambda: (0, 0)),
            pl.BlockSpec((1, E), lambda: (0, 0)),
            pl.BlockSpec(memory_space=pltpu.SMEM),
        ],
        out_specs=[
            pl.BlockSpec((B, E), lambda: (0, 0)),
            pl.BlockSpec((2 * B, 1), lambda: (0, 0)),
            pl.BlockSpec((NW, 2, B // NW), lambda: (0, 0, 0)),
            pl.BlockSpec((B, 1), lambda: (0, 0)),
            pl.BlockSpec((B, 1), lambda: (0, 0)),
            pl.BlockSpec((B, 1), lambda: (0, 0)),
            pl.BlockSpec((B, 1), lambda: (0, 0)),
            pl.BlockSpec((32, 1), lambda: (0, 0)),
        ],
        out_shape=[
            jax.ShapeDtypeStruct((B, E), jnp.float32),
            jax.ShapeDtypeStruct((2 * B, 1), jnp.int32),
            jax.ShapeDtypeStruct((NW, 2, B // NW), jnp.int32),
            jax.ShapeDtypeStruct((B, 1), jnp.float32),
            jax.ShapeDtypeStruct((B, 1), jnp.float32),
            jax.ShapeDtypeStruct((B, 1), jnp.float32),
            jax.ShapeDtypeStruct((B, 1), jnp.float32),
            jax.ShapeDtypeStruct((32, 1), jnp.int32),
        ],
    )(x, gate_w1, gate_b1.reshape(1, H2), gate_w2, gate_b2.reshape(1, E),
      alpha)


def _sc_scatter_rows(x, idx3, n_out):
    B, D = x.shape
    tpw = B // NW
    mesh = plsc.VectorSubcoreMesh(core_axis_name="c", subcore_axis_name="s")

    @functools.partial(
        pl.kernel, mesh=mesh,
        out_type=jax.ShapeDtypeStruct((n_out, D), jnp.float32),
        scratch_types=[
            pltpu.VMEM((2, tpw), jnp.int32),
            pltpu.VMEM((tpw, D), jnp.float32),
            pltpu.SemaphoreType.DMA,
            pltpu.SemaphoreType.DMA,
        ],
    )
    def k(x_hbm, idx_hbm, out_hbm, idx_v, buf, sem0, sem1):
        wid = lax.axis_index("s") * NC + lax.axis_index("c")
        pltpu.sync_copy(idx_hbm.at[wid], idx_v)
        pltpu.sync_copy(x_hbm.at[pl.ds(wid * tpw, tpw)], buf)
        c0 = pltpu.async_copy(buf, out_hbm.at[idx_v.at[0]], sem0)
        c1 = pltpu.async_copy(buf, out_hbm.at[idx_v.at[1]], sem1)
        c0.wait()
        c1.wait()

    return k(x, idx3)


def _sc_gather(table, idx, chunk):
    n_rows = idx.shape[0]
    D = table.shape[1]
    rows_per_tile = n_rows // NW
    n_chunks = rows_per_tile // chunk
    mesh = plsc.VectorSubcoreMesh(core_axis_name="c", subcore_axis_name="s")

    @functools.partial(
        pl.kernel, mesh=mesh,
        out_type=jax.ShapeDtypeStruct((n_rows, D), jnp.float32),
        scratch_types=[
            pltpu.VMEM((rows_per_tile,), jnp.int32),
            pltpu.VMEM((chunk, D), jnp.float32),
            pltpu.VMEM((chunk, D), jnp.float32),
            pltpu.SemaphoreType.DMA,
            pltpu.SemaphoreType.DMA,
        ],
    )
    def k(table_hbm, idx_hbm, out_hbm, idx_v, buf0, buf1, sem_g, sem_s):
        wid = lax.axis_index("s") * NC + lax.axis_index("c")
        base = wid * rows_per_tile
        bufs = (buf0, buf1)
        pltpu.sync_copy(idx_hbm.at[pl.ds(base, rows_per_tile)], idx_v)
        gathers = []
        stores = []
        gathers.append(pltpu.async_copy(
            table_hbm.at[idx_v.at[pl.ds(0, chunk)]], bufs[0], sem_g))
        for c in range(n_chunks):
            gathers[c].wait()
            if c + 1 < n_chunks:
                if c >= 1:
                    stores[c - 1].wait()
                gathers.append(pltpu.async_copy(
                    table_hbm.at[idx_v.at[pl.ds((c + 1) * chunk, chunk)]],
                    bufs[(c + 1) % 2], sem_g))
            stores.append(pltpu.async_copy(
                bufs[c % 2], out_hbm.at[pl.ds(base + c * chunk, chunk)],
                sem_s))
        stores[n_chunks - 2].wait()
        stores[n_chunks - 1].wait()

    return k(table, idx)


def _expert_body(bn_ref, xs_ref,
                 ew1_ref, eb1_ref, ew2_ref, eb2_ref, pw_ref, zs_ref):
    g = pl.program_id(0)

    @pl.when(g < bn_ref[pl.num_programs(0), 0])
    def _compute():
        xb = xs_ref[...]
        h1 = jnp.dot(xb, ew1_ref[0], preferred_element_type=jnp.float32)
        h1 = h1 + eb1_ref[0]
        h1 = h1 * jax.nn.sigmoid(h1)
        eo = jnp.dot(h1, ew2_ref[0], preferred_element_type=jnp.float32)
        eo = eo + eb2_ref[0]
        zs_ref[...] = jnp.dot(eo, pw_ref[0], preferred_element_type=jnp.float32)

    @pl.when(g >= bn_ref[pl.num_programs(0), 0])
    def _zero():
        zs_ref[...] = jnp.zeros_like(zs_ref)


def _expert_call(xs, expert_w1, expert_b1, expert_w2, expert_b2, proj_w,
                 benb, g_max):
    G_CAP, H = xs.shape
    E, _, F = expert_w1.shape
    grid_spec = pltpu.PrefetchScalarGridSpec(
        num_scalar_prefetch=1,
        grid=(g_max,),
        in_specs=[
            pl.BlockSpec((BM, H), lambda g, bn: (g, 0)),
            pl.BlockSpec((1, H, F), lambda g, bn: (bn[g, 0], 0, 0)),
            pl.BlockSpec((1, 1, F), lambda g, bn: (bn[g, 0], 0, 0)),
            pl.BlockSpec((1, F, H), lambda g, bn: (bn[g, 0], 0, 0)),
            pl.BlockSpec((1, 1, H), lambda g, bn: (bn[g, 0], 0, 0)),
            pl.BlockSpec((1, H, H), lambda g, bn: (bn[g, 0], 0, 0)),
        ],
        out_specs=pl.BlockSpec(
            (BM, H),
            lambda g, bn: (jnp.where(g < bn[g_max, 0], g, g_max - 1), 0)),
    )
    return pl.pallas_call(
        _expert_body,
        grid_spec=grid_spec,
        out_shape=jax.ShapeDtypeStruct((G_CAP, H), jnp.float32),
    )(benb, xs,
      expert_w1, expert_b1.reshape(E, 1, F), expert_w2,
      expert_b2.reshape(E, 1, H), proj_w)


def _combine_body(g0_ref, g1_ref, x_ref, wt0_ref, wt1_ref, s1_ref, s2_ref,
                  out_ref):
    acc = wt0_ref[...] * g0_ref[...] + wt1_ref[...] * g1_ref[...]
    out_ref[...] = s1_ref[...] * acc + s2_ref[...] * x_ref[...]


def _combine_call(gath, x, wt0, wt1, s1, s2):
    B, H = x.shape
    BMc = 512
    MB = B // BMc
    return pl.pallas_call(
        _combine_body,
        grid=(MB,),
        in_specs=[
            pl.BlockSpec((BMc, H), lambda mb: (mb, 0)),
            pl.BlockSpec((BMc, H), lambda mb, _MB=MB: (mb + _MB, 0)),
            pl.BlockSpec((BMc, H), lambda mb: (mb, 0)),
            pl.BlockSpec((BMc, 1), lambda mb: (mb, 0)),
            pl.BlockSpec((BMc, 1), lambda mb: (mb, 0)),
            pl.BlockSpec((BMc, 1), lambda mb: (mb, 0)),
            pl.BlockSpec((BMc, 1), lambda mb: (mb, 0)),
        ],
        out_specs=pl.BlockSpec((BMc, H), lambda mb: (mb, 0)),
        out_shape=jax.ShapeDtypeStruct((B, H), jnp.float32),
    )(gath, gath, x, wt0, wt1, s1, s2)


def kernel(x, gate_w1, gate_b1, gate_w2, gate_b2,
           expert_w1, expert_b1, expert_w2, expert_b2, proj_w, blend):
    B, H = x.shape
    E = gate_w2.shape[1]
    G_MAX = (2 * B) // BM + E + 1
    G_CAP = G_MAX * BM

    alpha = jax.nn.sigmoid(blend).reshape(1, 1).astype(jnp.float32)
    (gate_weights, destcat, idx3, wt0, wt1, s1, s2, benb) = _route_call(
        x, gate_w1, gate_b1, gate_w2, gate_b2, alpha, G_MAX, G_CAP)

    destflat = destcat.reshape(2 * B)

    xs = _sc_scatter_rows(x, idx3, G_CAP)
    zs = _expert_call(xs, expert_w1, expert_b1, expert_w2, expert_b2, proj_w,
                      benb, G_MAX)
    gath = _sc_gather(zs, destflat, chunk=32)
    out = _combine_call(gath, x, wt0, wt1, s1, s2)
    return out, gate_weights
